# Initial kernel scaffold; baseline (speedup 1.0000x reference)
import functools

import jax
import jax.numpy as jnp
from jax import lax
from jax.experimental import pallas as pl
from jax.experimental.pallas import tpu as pltpu
from jax.experimental.pallas import tpu_sc as plsc

N = 1000000
D = 32
B = 16384
NC = 2
NS = 16
NW = NC * NS
BPW = B // NW
CH = 512
COPY_BLK = 20000
N_BLKS = N // COPY_BLK

_sc_mesh = functools.partial(
    plsc.VectorSubcoreMesh, core_axis_name="c", subcore_axis_name="s"
)


@functools.partial(
    pl.kernel,
    mesh=_sc_mesh(),
    out_type=jax.ShapeDtypeStruct((B, D), jnp.float32),
    scratch_types=[
        pltpu.VMEM((BPW,), jnp.int32),
        pltpu.VMEM((BPW, D), jnp.float32),
        pltpu.SemaphoreType.DMA,
    ],
)
def _gather_rows(traj_hbm, ids_hbm, out_hbm, idx_v, rows_v, sem):
    wid = lax.axis_index("s") * NC + lax.axis_index("c")
    base = wid * BPW
    pltpu.sync_copy(ids_hbm.at[pl.ds(base, BPW)], idx_v)
    pltpu.async_copy(traj_hbm.at[idx_v], rows_v, sem).wait()
    pltpu.sync_copy(rows_v, out_hbm.at[pl.ds(base, BPW)])


def _gru_copy_body(traj_ref, msg_ref, h_ref, wih_ref, whh_ref, bih_ref,
                   bhh_ref, out_ref, newh_ref):
    out_ref[...] = traj_ref[...]

    @pl.when(pl.program_id(0) == 0)
    def _():
        x = msg_ref[...]
        h = h_ref[...]
        gi = lax.dot_general(x, wih_ref[...], (((1,), (1,)), ((), ())),
                             preferred_element_type=jnp.float32) + bih_ref[...]
        gh = lax.dot_general(h, whh_ref[...], (((1,), (1,)), ((), ())),
                             preferred_element_type=jnp.float32) + bhh_ref[...]
        r = jax.nn.sigmoid(gi[:, 0:D] + gh[:, 0:D])
        z = jax.nn.sigmoid(gi[:, D:2 * D] + gh[:, D:2 * D])
        n = jnp.tanh(gi[:, 2 * D:3 * D] + r * gh[:, 2 * D:3 * D])
        newh_ref[...] = (1.0 - z) * n + z * h


def _gru_copy(traj, messages, h, W_ih, W_hh, b_ih, b_hh):
    const = lambda shape: pl.BlockSpec(shape, lambda i: (0,) * len(shape))
    return pl.pallas_call(
        _gru_copy_body,
        grid=(N_BLKS,),
        in_specs=[
            pl.BlockSpec((COPY_BLK, D), lambda i: (i, 0)),
            const((B, D)),
            const((B, D)),
            const((3 * D, D)),
            const((3 * D, D)),
            const((1, 3 * D)),
            const((1, 3 * D)),
        ],
        out_specs=[
            pl.BlockSpec((COPY_BLK, D), lambda i: (i, 0)),
            const((B, D)),
        ],
        out_shape=[
            jax.ShapeDtypeStruct((N, D), jnp.float32),
            jax.ShapeDtypeStruct((B, D), jnp.float32),
        ],
    )(traj, messages, h, W_ih, W_hh,
      b_ih.reshape(1, 3 * D), b_hh.reshape(1, 3 * D))


@functools.partial(
    pl.kernel,
    mesh=_sc_mesh(),
    out_type=(),
    scratch_types=[
        pltpu.VMEM((CH,), jnp.int32),
        pltpu.VMEM((CH, D), jnp.float32),
        pltpu.SemaphoreType.DMA,
    ],
)
def _scatter_rows(ids2d_hbm, newh_hbm, out_ref, idx_v, rows_v, sem):
    wid = lax.axis_index("s") * NC + lax.axis_index("c")

    @pl.when(wid == 0)
    def _():
        def body(k, carry):
            pltpu.sync_copy(ids2d_hbm.at[k], idx_v)
            pltpu.sync_copy(newh_hbm.at[pl.ds(k * CH, CH)], rows_v)
            pltpu.async_copy(rows_v, out_ref.at[idx_v], sem).wait()
            return carry

        lax.fori_loop(0, B // CH, body, 0)


def kernel(node_ids, messages, node_trajectory, W_ih, W_hh, b_ih, b_hh):
    h = _gather_rows(node_trajectory, node_ids)
    out_copy, new_h = _gru_copy(node_trajectory, messages, h,
                                W_ih, W_hh, b_ih, b_hh)
    out = jax.new_ref(out_copy)
    _scatter_rows(node_ids.reshape(NW, CH), new_h, out)
    return jax.freeze(out)


# speedup vs baseline: 1.8871x; 1.8871x over previous
import functools

import jax
import jax.numpy as jnp
from jax import lax
from jax.experimental import pallas as pl
from jax.experimental.pallas import tpu as pltpu
from jax.experimental.pallas import tpu_sc as plsc

N = 1000000
D = 32
B = 16384
NC = 2
NS = 16
NW = NC * NS
BPW = B // NW
CH = 512
GRU_BLK = 2048

_sc_mesh = functools.partial(
    plsc.VectorSubcoreMesh, core_axis_name="c", subcore_axis_name="s",
    num_cores=NC, num_subcores=NS,
)
_sc_params = pltpu.CompilerParams(use_tc_tiling_on_sc=False)


@functools.partial(
    pl.kernel,
    mesh=_sc_mesh(),
    out_type=jax.ShapeDtypeStruct((B, D), jnp.float32),
    scratch_types=[
        pltpu.VMEM((BPW,), jnp.int32),
        pltpu.VMEM((BPW, D), jnp.float32),
        pltpu.SemaphoreType.DMA,
    ],
    compiler_params=_sc_params,
)
def _gather_rows(table_ref, ids_hbm, out_hbm, idx_v, rows_v, sem):
    wid = lax.axis_index("s") * NC + lax.axis_index("c")
    base = wid * BPW
    pltpu.sync_copy(ids_hbm.at[pl.ds(base, BPW)], idx_v)
    pltpu.async_copy(table_ref.at[idx_v], rows_v, sem).wait()
    pltpu.sync_copy(rows_v, out_hbm.at[pl.ds(base, BPW)])


def _gru_body(msg_ref, h_ref, wih_ref, whh_ref, bih_ref, bhh_ref, newh_ref):
    x = msg_ref[...]
    h = h_ref[...]
    gi = lax.dot_general(x, wih_ref[...], (((1,), (1,)), ((), ())),
                         preferred_element_type=jnp.float32) + bih_ref[...]
    gh = lax.dot_general(h, whh_ref[...], (((1,), (1,)), ((), ())),
                         preferred_element_type=jnp.float32) + bhh_ref[...]
    r = jax.nn.sigmoid(gi[:, 0:D] + gh[:, 0:D])
    z = jax.nn.sigmoid(gi[:, D:2 * D] + gh[:, D:2 * D])
    n = jnp.tanh(gi[:, 2 * D:3 * D] + r * gh[:, 2 * D:3 * D])
    newh_ref[...] = (1.0 - z) * n + z * h


def _gru(messages, h, W_ih, W_hh, b_ih, b_hh):
    const = lambda shape: pl.BlockSpec(shape, lambda i: (0,) * len(shape))
    return pl.pallas_call(
        _gru_body,
        grid=(B // GRU_BLK,),
        in_specs=[
            pl.BlockSpec((GRU_BLK, D), lambda i: (i, 0)),
            pl.BlockSpec((GRU_BLK, D), lambda i: (i, 0)),
            const((3 * D, D)),
            const((3 * D, D)),
            const((1, 3 * D)),
            const((1, 3 * D)),
        ],
        out_specs=pl.BlockSpec((GRU_BLK, D), lambda i: (i, 0)),
        out_shape=jax.ShapeDtypeStruct((B, D), jnp.float32),
    )(messages, h, W_ih, W_hh, b_ih.reshape(1, 3 * D), b_hh.reshape(1, 3 * D))


@functools.partial(
    pl.kernel,
    mesh=_sc_mesh(),
    out_type=(),
    scratch_types=[
        pltpu.VMEM((CH,), jnp.int32),
        pltpu.VMEM((CH, D), jnp.float32),
        pltpu.SemaphoreType.DMA,
    ],
    compiler_params=_sc_params,
)
def _scatter_rows(ids2d_hbm, newh_hbm, table_ref, idx_v, rows_v, sem):
    wid = lax.axis_index("s") * NC + lax.axis_index("c")

    @pl.when(wid == 0)
    def _():
        def body(k, carry):
            pltpu.sync_copy(ids2d_hbm.at[k], idx_v)
            pltpu.sync_copy(newh_hbm.at[pl.ds(k * CH, CH)], rows_v)
            pltpu.async_copy(rows_v, table_ref.at[idx_v], sem).wait()
            return carry

        lax.fori_loop(0, B // CH, body, 0)


def kernel(node_ids, messages, node_trajectory, W_ih, W_hh, b_ih, b_hh):
    table = jax.new_ref(node_trajectory)
    h = _gather_rows(table, node_ids)
    new_h = _gru(messages, h, W_ih, W_hh, b_ih, b_hh)
    _scatter_rows(node_ids.reshape(NW, CH), new_h, table)
    return jax.freeze(table)


# speedup vs baseline: 3.0432x; 1.6126x over previous
import functools

import jax
import jax.numpy as jnp
from jax import lax
from jax.experimental import pallas as pl
from jax.experimental.pallas import tpu as pltpu
from jax.experimental.pallas import tpu_sc as plsc

N = 1000000
D = 32
B = 16384
NC = 2
NS = 16
NW = NC * NS
BK = 512
NMAIN = 999936
TAIL = N - NMAIN
RSZ = 31232
RSZ_LAST = NMAIN - (NW - 1) * RSZ
GRU_BLK = 2048

_sc_mesh = functools.partial(
    plsc.VectorSubcoreMesh, core_axis_name="c", subcore_axis_name="s",
    num_cores=NC, num_subcores=NS,
)
_sc_params = pltpu.CompilerParams(needs_layout_passes=False)


def _compact_in_range(ids_v, pk_v, base, size, iota):
    def body(c, o):
        idc = ids_v[pl.ds(c * 16, 16)]
        loc = idc - base
        m = (loc >= 0) & (loc < size)
        n = plsc.all_reduce_population_count(m)[0]

        @pl.when(n > 0)
        def _():
            packed = loc * B + (c * 16 + iota)
            plsc.store_compressed(pk_v.at[pl.ds(o, 16)], packed, mask=m)
        return o + n
    return lax.fori_loop(0, B // 16, body, 0)


def _block_matches(pk_v, bl_v, nw, lo, ext, iota):
    def body(g, ob):
        p16 = pk_v[pl.ds(g * 16, 16)]
        cl = lax.shift_right_logical(p16, 14)
        m = (cl >= lo) & (cl < lo + ext) & ((g * 16 + iota) < nw)
        n = plsc.all_reduce_population_count(m)[0]

        @pl.when(n > 0)
        def _():
            plsc.store_compressed(bl_v.at[pl.ds(ob, 16)], p16 - lo * B,
                                  mask=m)
        return ob + n
    return lax.fori_loop(0, (nw + 15) // 16, body, 0)


@functools.partial(
    pl.kernel,
    mesh=_sc_mesh(),
    out_type=jax.ShapeDtypeStruct((B * D,), jnp.float32),
    scratch_types=[
        pltpu.VMEM((B,), jnp.int32),
        pltpu.VMEM((B + 16,), jnp.int32),
        pltpu.VMEM((B + 16,), jnp.int32),
        pltpu.VMEM((32, BK), jnp.float32),
        pltpu.VMEM((32,), jnp.float32),
        pltpu.SemaphoreType.DMA,
    ],
    compiler_params=_sc_params,
)
def _gather_cols(tableT_hbm, ids_hbm, h_hbm, ids_v, pk_v, bl_v, buf, col_v,
                 sem):
    wid = lax.axis_index("s") * NC + lax.axis_index("c")
    base = wid * RSZ
    size = jnp.where(wid == NW - 1, RSZ_LAST, RSZ)
    iota = lax.iota(jnp.int32, 16)
    pltpu.sync_copy(ids_hbm, ids_v)
    nw = _compact_in_range(ids_v, pk_v, base, size, iota)

    def do_block(blk, ext):
        lo = blk * BK
        nb = _block_matches(pk_v, bl_v, nw, lo, ext, iota)

        @pl.when(nb > 0)
        def _():
            pltpu.sync_copy(tableT_hbm.at[:, pl.ds(base + lo, ext)],
                            buf.at[:, pl.ds(0, ext)])

            def grp(g, c):
                p16 = bl_v[pl.ds(g * 16, 16)]
                c16 = lax.shift_right_logical(p16, 14)
                i16 = p16 & (B - 1)
                left = nb - g * 16
                for j in range(16):
                    @pl.when(j < left)
                    def _():
                        cj = c16[j]
                        ij = i16[j]
                        cjf = jnp.full((16,), cj, jnp.int32)
                        lovals = plsc.load_gather(buf, [iota, cjf])
                        hivals = plsc.load_gather(buf, [iota + 16, cjf])
                        col_v[pl.ds(0, 16)] = lovals
                        col_v[pl.ds(16, 16)] = hivals
                        pltpu.async_copy(
                            col_v, h_hbm.at[pl.ds(ij * D, D)], sem).wait()
                return c
            lax.fori_loop(0, (nb + 15) // 16, grp, 0)

    nfull = jnp.where(wid == NW - 1, RSZ_LAST // BK, RSZ // BK)

    def fb(blk, c):
        do_block(blk, BK)
        return c
    lax.fori_loop(0, nfull, fb, 0)


def _gru_body(msg_ref, h_ref, wih_ref, whh_ref, bih_ref, bhh_ref, newh_ref):
    x = msg_ref[...]
    h = h_ref[...]
    gi = lax.dot_general(x, wih_ref[...], (((1,), (1,)), ((), ())),
                         preferred_element_type=jnp.float32) + bih_ref[...]
    gh = lax.dot_general(h, whh_ref[...], (((1,), (1,)), ((), ())),
                         preferred_element_type=jnp.float32) + bhh_ref[...]
    r = jax.nn.sigmoid(gi[:, 0:D] + gh[:, 0:D])
    z = jax.nn.sigmoid(gi[:, D:2 * D] + gh[:, D:2 * D])
    n = jnp.tanh(gi[:, 2 * D:3 * D] + r * gh[:, 2 * D:3 * D])
    newh_ref[...] = (1.0 - z) * n + z * h


def _gru(messages, h, W_ih, W_hh, b_ih, b_hh):
    const = lambda shape: pl.BlockSpec(shape, lambda i: (0,) * len(shape))
    return pl.pallas_call(
        _gru_body,
        grid=(B // GRU_BLK,),
        in_specs=[
            pl.BlockSpec((GRU_BLK, D), lambda i: (i, 0)),
            pl.BlockSpec((GRU_BLK, D), lambda i: (i, 0)),
            const((3 * D, D)),
            const((3 * D, D)),
            const((1, 3 * D)),
            const((1, 3 * D)),
        ],
        out_specs=pl.BlockSpec((GRU_BLK, D), lambda i: (i, 0)),
        out_shape=jax.ShapeDtypeStruct((B, D), jnp.float32),
    )(messages, h, W_ih, W_hh, b_ih.reshape(1, 3 * D), b_hh.reshape(1, 3 * D))


@functools.partial(
    pl.kernel,
    mesh=_sc_mesh(),
    out_type=jax.ShapeDtypeStruct((D, N), jnp.float32),
    scratch_types=[
        pltpu.VMEM((B,), jnp.int32),
        pltpu.VMEM((B + 16,), jnp.int32),
        pltpu.VMEM((B + 16,), jnp.int32),
        pltpu.VMEM((32, BK), jnp.float32),
        pltpu.VMEM((16, D), jnp.float32),
        pltpu.SemaphoreType.DMA,
    ],
    compiler_params=_sc_params,
)
def _copy_scatter(tableT_hbm, ids_hbm, newh_hbm, outT_hbm, ids_v, pk_v, bl_v,
                  buf, rows_v, sem):
    wid = lax.axis_index("s") * NC + lax.axis_index("c")
    base = wid * RSZ
    size = jnp.where(wid == NW - 1, RSZ_LAST, RSZ)
    iota = lax.iota(jnp.int32, 16)
    pltpu.sync_copy(ids_hbm, ids_v)
    nw = _compact_in_range(ids_v, pk_v, base, size, iota)

    def do_block(blk, ext):
        lo = blk * BK
        nb = _block_matches(pk_v, bl_v, nw, lo, ext, iota)
        pltpu.sync_copy(tableT_hbm.at[:, pl.ds(base + lo, ext)],
                        buf.at[:, pl.ds(0, ext)])

        @pl.when(nb > 0)
        def _():
            def grp(g, c):
                p16 = bl_v[pl.ds(g * 16, 16)]
                c16 = lax.shift_right_logical(p16, 14)
                i16 = p16 & (B - 1)
                left = nb - g * 16
                for j in range(16):
                    @pl.when(j < left)
                    def _():
                        pltpu.async_copy(
                            newh_hbm.at[pl.ds(i16[j] * D, D)],
                            rows_v.at[j], sem).wait()
                for j in range(16):
                    @pl.when(j < left)
                    def _():
                        cjf = jnp.full((16,), c16[j], jnp.int32)
                        plsc.store_scatter(buf, [iota, cjf],
                                           rows_v[j, pl.ds(0, 16)])
                        plsc.store_scatter(buf, [iota + 16, cjf],
                                           rows_v[j, pl.ds(16, 16)])
                return c
            lax.fori_loop(0, (nb + 15) // 16, grp, 0)

        pltpu.sync_copy(buf.at[:, pl.ds(0, ext)],
                        outT_hbm.at[:, pl.ds(base + lo, ext)])

    nfull = jnp.where(wid == NW - 1, RSZ_LAST // BK, RSZ // BK)

    def fb(blk, c):
        do_block(blk, BK)
        return c
    lax.fori_loop(0, nfull, fb, 0)


def _tail_body(alias_ref, tailT_ref, ids_ref, msg_ref, wih_ref, whh_ref,
               bih_ref, bhh_ref, out_ref):
    del alias_ref
    out_ref[...] = tailT_ref[...]
    ids2 = ids_ref[...]
    iota2 = (lax.broadcasted_iota(jnp.int32, (128, 128), 0) * 128 +
             lax.broadcasted_iota(jnp.int32, (128, 128), 1))
    for c in range(TAIL):
        m = ids2 == (NMAIN + c)
        win = jnp.max(jnp.where(m, iota2, -1))

        @pl.when(win >= 0)
        def _(c=c, win=win):
            mrow = msg_ref[pl.ds(win, 1), :]
            hcol = out_ref[:, pl.ds(c, 1)]
            gi = jnp.sum(wih_ref[...] * mrow, axis=1,
                         keepdims=True) + bih_ref[...]
            gh = jnp.sum(whh_ref[...] * hcol.reshape(1, D), axis=1,
                         keepdims=True) + bhh_ref[...]
            r = jax.nn.sigmoid(gi[0:D] + gh[0:D])
            z = jax.nn.sigmoid(gi[D:2 * D] + gh[D:2 * D])
            n = jnp.tanh(gi[2 * D:3 * D] + r * gh[2 * D:3 * D])
            out_ref[:, pl.ds(c, 1)] = (1.0 - z) * n + z * hcol


def _tail_fix(outT, tableT, node_ids, messages, W_ih, W_hh, b_ih, b_hh):
    const = lambda shape: pl.BlockSpec(shape, lambda i: (0,) * len(shape))
    return pl.pallas_call(
        _tail_body,
        grid=(1,),
        in_specs=[
            pl.BlockSpec(memory_space=pl.ANY),
            pl.BlockSpec((D, 128), lambda i: (0, NMAIN // 128)),
            const((128, 128)),
            const((B, D)),
            const((3 * D, D)),
            const((3 * D, D)),
            const((3 * D, 1)),
            const((3 * D, 1)),
        ],
        out_specs=pl.BlockSpec((D, 128), lambda i: (0, NMAIN // 128)),
        out_shape=jax.ShapeDtypeStruct((D, N), jnp.float32),
        input_output_aliases={0: 0},
    )(outT, tableT, node_ids.reshape(128, 128), messages, W_ih, W_hh,
      b_ih.reshape(3 * D, 1), b_hh.reshape(3 * D, 1))


def kernel(node_ids, messages, node_trajectory, W_ih, W_hh, b_ih, b_hh):
    tableT = node_trajectory.T
    h = _gather_cols(tableT, node_ids).reshape(B, D)
    new_h = _gru(messages, h, W_ih, W_hh, b_ih, b_hh)
    outT = _copy_scatter(tableT, node_ids, new_h.reshape(B * D))
    outT = _tail_fix(outT, tableT, node_ids, messages, W_ih, W_hh, b_ih, b_hh)
    return outT.T


# speedup vs baseline: 6.4029x; 2.1040x over previous
import functools

import jax
import jax.numpy as jnp
from jax import lax
from jax.experimental import pallas as pl
from jax.experimental.pallas import tpu as pltpu
from jax.experimental.pallas import tpu_sc as plsc

N = 1000000
D = 32
B = 16384
NC = 2
NS = 16
NW = NC * NS
BK = 512
NMAIN = 999936
TAIL = N - NMAIN
RSZ = 31232
RSZ_LAST = NMAIN - (NW - 1) * RSZ
GRU_BLK = 2048

_sc_mesh = functools.partial(
    plsc.VectorSubcoreMesh, core_axis_name="c", subcore_axis_name="s",
    num_cores=NC, num_subcores=NS,
)
_sc_params = pltpu.CompilerParams(needs_layout_passes=False)


def _compact_in_range(ids_v, pk_v, base, size, iota):
    def body(c, o):
        idc = ids_v[pl.ds(c * 16, 16)]
        loc = idc - base
        m = (loc >= 0) & (loc < size)
        n = plsc.all_reduce_population_count(m)[0]

        @pl.when(n > 0)
        def _():
            packed = loc * B + (c * 16 + iota)
            plsc.store_compressed(pk_v.at[pl.ds(o, 16)], packed, mask=m)
        return o + n
    return lax.fori_loop(0, B // 16, body, 0)


def _block_matches(pk_v, bl_v, nw, lo, ext, iota):
    def body(g, ob):
        p16 = pk_v[pl.ds(g * 16, 16)]
        cl = lax.shift_right_logical(p16, 14)
        m = (cl >= lo) & (cl < lo + ext) & ((g * 16 + iota) < nw)
        n = plsc.all_reduce_population_count(m)[0]

        @pl.when(n > 0)
        def _():
            plsc.store_compressed(bl_v.at[pl.ds(ob, 16)], p16 - lo * B,
                                  mask=m)
        return ob + n
    return lax.fori_loop(0, (nw + 15) // 16, body, 0)


@functools.partial(
    pl.kernel,
    mesh=_sc_mesh(),
    out_type=jax.ShapeDtypeStruct((B * D,), jnp.float32),
    scratch_types=[
        pltpu.VMEM((B,), jnp.int32),
        pltpu.VMEM((B + 16,), jnp.int32),
        pltpu.VMEM((B + 16,), jnp.int32),
        pltpu.VMEM((2, 32, BK), jnp.float32),
        pltpu.VMEM((16, D), jnp.float32),
        pltpu.SemaphoreType.DMA,
        pltpu.SemaphoreType.DMA,
    ],
    compiler_params=_sc_params,
)
def _gather_cols(tableT_hbm, ids_hbm, h_hbm, ids_v, pk_v, bl_v, buf2, colg,
                 sem_in, sem_h):
    wid = lax.axis_index("s") * NC + lax.axis_index("c")
    base = wid * RSZ
    size = jnp.where(wid == NW - 1, RSZ_LAST, RSZ)
    iota = lax.iota(jnp.int32, 16)
    pltpu.sync_copy(ids_hbm, ids_v)
    nw = _compact_in_range(ids_v, pk_v, base, size, iota)
    nfull = jnp.where(wid == NW - 1, RSZ_LAST // BK, RSZ // BK)

    def start_in(blk, buf):
        pltpu.async_copy(tableT_hbm.at[:, pl.ds(base + blk * BK, BK)], buf,
                         sem_in)

    def wait_in(buf):
        pltpu.make_async_copy(tableT_hbm.at[:, pl.ds(0, BK)], buf,
                              sem_in).wait()

    def process(blk, buf):
        nb = _block_matches(pk_v, bl_v, nw, blk * BK, BK, iota)

        @pl.when(nb > 0)
        def _():
            def grp(g, c):
                p16 = bl_v[pl.ds(g * 16, 16)]
                c16 = lax.shift_right_logical(p16, 14)
                i16 = p16 & (B - 1)
                left = nb - g * 16
                for j in range(16):
                    @pl.when(j < left)
                    def _():
                        cjf = jnp.full((16,), c16[j], jnp.int32)
                        colg[j, pl.ds(0, 16)] = plsc.load_gather(
                            buf, [iota, cjf])
                        colg[j, pl.ds(16, 16)] = plsc.load_gather(
                            buf, [iota + 16, cjf])
                        pltpu.async_copy(
                            colg.at[j], h_hbm.at[pl.ds(i16[j] * D, D)], sem_h)
                for j in range(16):
                    @pl.when(j < left)
                    def _():
                        pltpu.make_async_copy(
                            colg.at[j], h_hbm.at[pl.ds(0, D)], sem_h).wait()
                return c
            lax.fori_loop(0, (nb + 15) // 16, grp, 0)

    bufA, bufB = buf2.at[0], buf2.at[1]
    start_in(0, bufA)

    def body(k, c):
        def step(cur, nxt):
            @pl.when(k + 1 < nfull)
            def _():
                start_in(k + 1, nxt)
            wait_in(cur)
            process(k, cur)

        @pl.when(k % 2 == 0)
        def _():
            step(bufA, bufB)

        @pl.when(k % 2 == 1)
        def _():
            step(bufB, bufA)
        return c
    lax.fori_loop(0, nfull, body, 0)


def _gru_body(msg_ref, h_ref, wih_ref, whh_ref, bih_ref, bhh_ref, newh_ref):
    x = msg_ref[...]
    h = h_ref[...]
    gi = lax.dot_general(x, wih_ref[...], (((1,), (1,)), ((), ())),
                         preferred_element_type=jnp.float32) + bih_ref[...]
    gh = lax.dot_general(h, whh_ref[...], (((1,), (1,)), ((), ())),
                         preferred_element_type=jnp.float32) + bhh_ref[...]
    r = jax.nn.sigmoid(gi[:, 0:D] + gh[:, 0:D])
    z = jax.nn.sigmoid(gi[:, D:2 * D] + gh[:, D:2 * D])
    n = jnp.tanh(gi[:, 2 * D:3 * D] + r * gh[:, 2 * D:3 * D])
    newh_ref[...] = (1.0 - z) * n + z * h


def _gru(messages, h, W_ih, W_hh, b_ih, b_hh):
    const = lambda shape: pl.BlockSpec(shape, lambda i: (0,) * len(shape))
    return pl.pallas_call(
        _gru_body,
        grid=(B // GRU_BLK,),
        in_specs=[
            pl.BlockSpec((GRU_BLK, D), lambda i: (i, 0)),
            pl.BlockSpec((GRU_BLK, D), lambda i: (i, 0)),
            const((3 * D, D)),
            const((3 * D, D)),
            const((1, 3 * D)),
            const((1, 3 * D)),
        ],
        out_specs=pl.BlockSpec((GRU_BLK, D), lambda i: (i, 0)),
        out_shape=jax.ShapeDtypeStruct((B, D), jnp.float32),
    )(messages, h, W_ih, W_hh, b_ih.reshape(1, 3 * D), b_hh.reshape(1, 3 * D))


@functools.partial(
    pl.kernel,
    mesh=_sc_mesh(),
    out_type=jax.ShapeDtypeStruct((D, N), jnp.float32),
    scratch_types=[
        pltpu.VMEM((B,), jnp.int32),
        pltpu.VMEM((B + 16,), jnp.int32),
        pltpu.VMEM((B + 16,), jnp.int32),
        pltpu.VMEM((2, 32, BK), jnp.float32),
        pltpu.VMEM((16, D), jnp.float32),
        pltpu.SemaphoreType.DMA,
        pltpu.SemaphoreType.DMA,
        pltpu.SemaphoreType.DMA,
    ],
    compiler_params=_sc_params,
)
def _copy_scatter(tableT_hbm, ids_hbm, newh_hbm, outT_hbm, ids_v, pk_v, bl_v,
                  buf2, rows_v, sem_in, sem_out, sem_r):
    wid = lax.axis_index("s") * NC + lax.axis_index("c")
    base = wid * RSZ
    size = jnp.where(wid == NW - 1, RSZ_LAST, RSZ)
    iota = lax.iota(jnp.int32, 16)
    pltpu.sync_copy(ids_hbm, ids_v)
    nw = _compact_in_range(ids_v, pk_v, base, size, iota)
    nfull = jnp.where(wid == NW - 1, RSZ_LAST // BK, RSZ // BK)

    def start_in(blk, buf):
        pltpu.async_copy(tableT_hbm.at[:, pl.ds(base + blk * BK, BK)], buf,
                         sem_in)

    def wait_in(buf):
        pltpu.make_async_copy(tableT_hbm.at[:, pl.ds(0, BK)], buf,
                              sem_in).wait()

    def start_out(blk, buf):
        pltpu.async_copy(buf, outT_hbm.at[:, pl.ds(base + blk * BK, BK)],
                         sem_out)

    def wait_out(buf):
        pltpu.make_async_copy(buf, outT_hbm.at[:, pl.ds(0, BK)],
                              sem_out).wait()

    def process(blk, buf):
        nb = _block_matches(pk_v, bl_v, nw, blk * BK, BK, iota)

        @pl.when(nb > 0)
        def _():
            def grp(g, c):
                p16 = bl_v[pl.ds(g * 16, 16)]
                c16 = lax.shift_right_logical(p16, 14)
                i16 = p16 & (B - 1)
                left = nb - g * 16
                for j in range(16):
                    @pl.when(j < left)
                    def _():
                        pltpu.async_copy(
                            newh_hbm.at[pl.ds(i16[j] * D, D)],
                            rows_v.at[j], sem_r)
                for j in range(16):
                    @pl.when(j < left)
                    def _():
                        pltpu.make_async_copy(
                            newh_hbm.at[pl.ds(0, D)], rows_v.at[j],
                            sem_r).wait()
                for j in range(16):
                    @pl.when(j < left)
                    def _():
                        cjf = jnp.full((16,), c16[j], jnp.int32)
                        plsc.store_scatter(buf, [iota, cjf],
                                           rows_v[j, pl.ds(0, 16)])
                        plsc.store_scatter(buf, [iota + 16, cjf],
                                           rows_v[j, pl.ds(16, 16)])
                return c
            lax.fori_loop(0, (nb + 15) // 16, grp, 0)

    bufA, bufB = buf2.at[0], buf2.at[1]
    start_in(0, bufA)

    def body(k, c):
        def step(cur, nxt):
            @pl.when(k >= 1)
            def _():
                wait_out(nxt)
            @pl.when(k + 1 < nfull)
            def _():
                start_in(k + 1, nxt)
            wait_in(cur)
            process(k, cur)
            start_out(k, cur)

        @pl.when(k % 2 == 0)
        def _():
            step(bufA, bufB)

        @pl.when(k % 2 == 1)
        def _():
            step(bufB, bufA)
        return c
    lax.fori_loop(0, nfull, body, 0)

    @pl.when(nfull % 2 == 1)
    def _():
        wait_out(bufA)

    @pl.when(nfull % 2 == 0)
    def _():
        wait_out(bufB)


def _tail_body(alias_ref, tailT_ref, ids_ref, msg_ref, wih_ref, whh_ref,
               bih_ref, bhh_ref, out_ref):
    del alias_ref
    out_ref[...] = tailT_ref[...]
    ids2 = ids_ref[...]
    iota2 = (lax.broadcasted_iota(jnp.int32, (128, 128), 0) * 128 +
             lax.broadcasted_iota(jnp.int32, (128, 128), 1))
    for c in range(TAIL):
        m = ids2 == (NMAIN + c)
        win = jnp.max(jnp.where(m, iota2, -1))

        @pl.when(win >= 0)
        def _(c=c, win=win):
            mrow = msg_ref[pl.ds(win, 1), :]
            hcol = out_ref[:, pl.ds(c, 1)]
            gi = jnp.sum(wih_ref[...] * mrow, axis=1,
                         keepdims=True) + bih_ref[...]
            gh = jnp.sum(whh_ref[...] * hcol.reshape(1, D), axis=1,
                         keepdims=True) + bhh_ref[...]
            r = jax.nn.sigmoid(gi[0:D] + gh[0:D])
            z = jax.nn.sigmoid(gi[D:2 * D] + gh[D:2 * D])
            n = jnp.tanh(gi[2 * D:3 * D] + r * gh[2 * D:3 * D])
            out_ref[:, pl.ds(c, 1)] = (1.0 - z) * n + z * hcol


def _tail_fix(outT, tableT, node_ids, messages, W_ih, W_hh, b_ih, b_hh):
    const = lambda shape: pl.BlockSpec(shape, lambda i: (0,) * len(shape))
    return pl.pallas_call(
        _tail_body,
        grid=(1,),
        in_specs=[
            pl.BlockSpec(memory_space=pl.ANY),
            pl.BlockSpec((D, 128), lambda i: (0, NMAIN // 128)),
            const((128, 128)),
            const((B, D)),
            const((3 * D, D)),
            const((3 * D, D)),
            const((3 * D, 1)),
            const((3 * D, 1)),
        ],
        out_specs=pl.BlockSpec((D, 128), lambda i: (0, NMAIN // 128)),
        out_shape=jax.ShapeDtypeStruct((D, N), jnp.float32),
        input_output_aliases={0: 0},
    )(outT, tableT, node_ids.reshape(128, 128), messages, W_ih, W_hh,
      b_ih.reshape(3 * D, 1), b_hh.reshape(3 * D, 1))


def kernel(node_ids, messages, node_trajectory, W_ih, W_hh, b_ih, b_hh):
    tableT = node_trajectory.T
    h = _gather_cols(tableT, node_ids).reshape(B, D)
    new_h = _gru(messages, h, W_ih, W_hh, b_ih, b_hh)
    outT = _copy_scatter(tableT, node_ids, new_h.reshape(B * D))
    outT = _tail_fix(outT, tableT, node_ids, messages, W_ih, W_hh, b_ih, b_hh)
    return outT.T
